# baseline (speedup 1.0000x reference)
import functools

import jax
import jax.numpy as jnp
from jax import lax
from jax.experimental import pallas as pl
from jax.experimental.pallas import tpu as pltpu
from jax.experimental.pallas import tpu_sc as plsc

B = 8
C = 192
HW = 224 * 224
ROWS = B * C
NUM_WORKERS = 32
R_PER_W = ROWS // NUM_WORKERS
W_PER_B = C // R_PER_W


def _body(x_hbm, perm_hbm, out_hbm, perm_v, buf_v, gsem, osem):
    cid = lax.axis_index("c")
    sid = lax.axis_index("s")
    wid = sid * 2 + cid

    pltpu.sync_copy(perm_hbm, perm_v)
    b = wid // W_PER_B
    c0 = (wid % W_PER_B) * R_PER_W
    base = wid * R_PER_W

    pvs = [perm_v[pl.ds(c0 + g * 16, 16)] for g in range(R_PER_W // 16)]

    NBUF = 4
    HALF = HW // 2
    NT = R_PER_W * 2

    def src_off(t):
        r = t // 2
        return (b * C + pvs[r // 16][r % 16]) * 2 + (t & 1)

    def gather(t, slot):
        return pltpu.async_copy(
            x_hbm.at[pl.ds(src_off(t), 1)],
            buf_v.at[pl.ds(slot, 1)],
            gsem,
        )

    def put(t, slot):
        return pltpu.async_copy(
            buf_v.at[pl.ds(slot, 1)],
            out_hbm.at[pl.ds(base * 2 + t, 1)],
            osem,
        )

    gh = {t: gather(t, t) for t in range(NBUF)}
    ph = {}
    for t in range(NT):
        s = t % NBUF
        if t >= NBUF:
            ph[t - NBUF].wait()
            gh[t] = gather(t, s)
        gh[t].wait()
        ph[t] = put(t, s)
    for t in range(NT - NBUF, NT):
        ph[t].wait()


_mesh = plsc.VectorSubcoreMesh(core_axis_name="c", subcore_axis_name="s")

_sc_permute = functools.partial(
    pl.kernel,
    mesh=_mesh,
    out_type=jax.ShapeDtypeStruct((ROWS * 2, HW // 2), jnp.float32),
    scratch_types=[
        pltpu.VMEM((C,), jnp.int32),
        pltpu.VMEM((4, HW // 2), jnp.float32),
        pltpu.SemaphoreType.DMA,
        pltpu.SemaphoreType.DMA,
    ],
)(_body)


def kernel(x, permutation):
    xf = x.reshape(ROWS * 2, HW // 2)
    perm = permutation.astype(jnp.int32)
    out = _sc_permute(xf, perm)
    return out.reshape(B, C, 224, 224)
